# Initial kernel scaffold; baseline (speedup 1.0000x reference)
import jax
import jax.numpy as jnp
from jax.experimental import pallas as pl


def kernel(HMatrix, adj, W1, b1, W2, b2):
    raise NotImplementedError("write your pallas kernel here")


# speedup vs baseline: 2.6022x; 2.6022x over previous
import functools

import jax
import jax.numpy as jnp
from jax.experimental import pallas as pl
from jax.experimental.pallas import tpu as pltpu

F32 = jnp.float32
BF16 = jnp.bfloat16


def _p1_body(a_ref, h_ref, w1_ref, a16_ref, d_ref, xhi_ref, xlo_ref):
    a = a_ref[...]
    a16_ref[...] = a.astype(BF16)
    deg = jnp.sum(a, axis=1, keepdims=True) + 1.0
    d = jax.lax.rsqrt(deg)
    d_ref[...] = d
    x1 = jnp.dot(h_ref[...], w1_ref[...], preferred_element_type=F32)
    xs = d * x1
    hi = xs.astype(BF16)
    xhi_ref[...] = hi
    xlo_ref[...] = (xs - hi.astype(F32)).astype(BF16)


def _layer1_body(br, a16_ref, xhi_ref, xlo_ref, d_ref, b1_ref, w2_ref,
                 x2hi_ref, x2lo_ref):
    a16 = a16_ref[...]
    y = (jnp.dot(a16, xhi_ref[...], preferred_element_type=F32)
         + jnp.dot(a16, xlo_ref[...], preferred_element_type=F32))
    i0 = pl.program_id(0) * br
    ident = (xhi_ref[pl.ds(i0, br), :].astype(F32)
             + xlo_ref[pl.ds(i0, br), :].astype(F32))
    d = d_ref[...]
    h = jax.nn.relu(d * (y + ident) + b1_ref[...])
    xs2 = d * jnp.dot(h, w2_ref[...], preferred_element_type=F32)
    hi = xs2.astype(BF16)
    x2hi_ref[...] = hi
    x2lo_ref[...] = (xs2 - hi.astype(F32)).astype(BF16)


def _layer2_body(br, a16_ref, xhi_ref, xlo_ref, d_ref, b2_ref, out_ref):
    a16 = a16_ref[...]
    y = (jnp.dot(a16, xhi_ref[...], preferred_element_type=F32)
         + jnp.dot(a16, xlo_ref[...], preferred_element_type=F32))
    i0 = pl.program_id(0) * br
    ident = (xhi_ref[pl.ds(i0, br), :].astype(F32)
             + xlo_ref[pl.ds(i0, br), :].astype(F32))
    o = d_ref[...] * (y + ident) + b2_ref[...]
    m = jnp.max(o, axis=1, keepdims=True)
    e = jnp.exp(o - m)
    out_ref[...] = e / jnp.sum(e, axis=1, keepdims=True)


def _pick_br(n):
    for br in (400, 200, 100, 50, 16, 8):
        if n % br == 0 and br % 16 == 0:
            return br
    return n


def kernel(HMatrix, adj, W1, b1, W2, b2):
    n, fin = HMatrix.shape
    hid = W1.shape[1]
    out_f = W2.shape[1]
    br = _pick_br(n)
    grid = (n // br,)
    params = pltpu.CompilerParams(dimension_semantics=("parallel",))

    b1r = b1.reshape(1, hid)
    b2r = b2.reshape(1, out_f)

    row_blk = lambda w: pl.BlockSpec((br, w), lambda i: (i, 0))
    full_blk = lambda h, w: pl.BlockSpec((h, w), lambda i: (0, 0))

    a16, d, xhi, xlo = pl.pallas_call(
        _p1_body,
        grid=grid,
        in_specs=[row_blk(n), row_blk(fin), full_blk(fin, hid)],
        out_specs=[row_blk(n), row_blk(1), row_blk(hid), row_blk(hid)],
        out_shape=[
            jax.ShapeDtypeStruct((n, n), BF16),
            jax.ShapeDtypeStruct((n, 1), F32),
            jax.ShapeDtypeStruct((n, hid), BF16),
            jax.ShapeDtypeStruct((n, hid), BF16),
        ],
        compiler_params=params,
    )(adj, HMatrix, W1)

    x2hi, x2lo = pl.pallas_call(
        functools.partial(_layer1_body, br),
        grid=grid,
        in_specs=[row_blk(n), full_blk(n, hid), full_blk(n, hid),
                  row_blk(1), full_blk(1, hid), full_blk(hid, out_f)],
        out_specs=[row_blk(out_f), row_blk(out_f)],
        out_shape=[
            jax.ShapeDtypeStruct((n, out_f), BF16),
            jax.ShapeDtypeStruct((n, out_f), BF16),
        ],
        compiler_params=params,
    )(a16, xhi, xlo, d, b1r, W2)

    out = pl.pallas_call(
        functools.partial(_layer2_body, br),
        grid=grid,
        in_specs=[row_blk(n), full_blk(n, out_f), full_blk(n, out_f),
                  row_blk(1), full_blk(1, out_f)],
        out_specs=row_blk(out_f),
        out_shape=jax.ShapeDtypeStruct((n, out_f), F32),
        compiler_params=params,
    )(a16, x2hi, x2lo, d, b2r)

    return out
